# Initial kernel scaffold; baseline (speedup 1.0000x reference)
import jax
import jax.numpy as jnp
from jax.experimental import pallas as pl


def kernel(X, linear_tables, dnn_tables, W_out):
    raise NotImplementedError("write your pallas kernel here")


# speedup vs baseline: 7.9631x; 7.9631x over previous
import functools

import jax
import jax.numpy as jnp
from jax import lax
from jax.experimental import pallas as pl
from jax.experimental.pallas import tpu as pltpu
from jax.experimental.pallas import tpu_sc as plsc

B = 16384
F = 26
V = 100000
D = 16

NC, NS, L = 2, 16, 16
NW = NC * NS
NB = B // NW
CB = 128
NSC = NB // CB
JG = CB // L


def _sc_body(x_hbm, lin_hbm, dnn_hbm, w_hbm, out_hbm,
             xl, idxb, linb, rows, wv, outl, sem_l, sem_d):
    wid = lax.axis_index("s") * NC + lax.axis_index("c")
    base = wid * NB

    pltpu.sync_copy(w_hbm, wv)
    pltpu.sync_copy(x_hbm.at[pl.ds(base * F, NB * F)], xl)

    iota = lax.iota(jnp.int32, L)

    def sub_chunk(sc, _):
        cbase = sc * CB

        def build_f(f, _):
            off = f * V
            for jc in range(JG):
                gidx = (iota + (cbase + jc * L)) * F + f
                vals = plsc.load_gather(xl, [gidx])
                idxb[pl.ds(f * CB + jc * L, L)] = vals + off
            return _
        lax.fori_loop(0, F, build_f, None)

        cp_l = pltpu.async_copy(lin_hbm.at[idxb], linb, sem_l)
        cp_d = pltpu.async_copy(dnn_hbm.at[idxb], rows, sem_d)
        cp_l.wait()
        cp_d.wait()

        def group(jg, _):
            jvec = iota + jg * L
            acc = jnp.zeros((L,), jnp.float32)
            for f in range(F):
                acc = acc + linb[pl.ds(f * CB + jg * L, L)]
                rvec = jvec + f * CB
                wrow = wv[f]
                for d in range(D):
                    col = plsc.load_gather(
                        rows, [rvec, jnp.full((L,), d, jnp.int32)])
                    acc = acc + col * wrow[d]
            outl[pl.ds(cbase + jg * L, L)] = 1.0 / (1.0 + jnp.exp(-acc))
            return _
        lax.fori_loop(0, JG, group, None)
        return _

    lax.fori_loop(0, NSC, sub_chunk, None)
    pltpu.sync_copy(outl, out_hbm.at[pl.ds(base, NB)])


@functools.partial(jax.jit, static_argnames=())
def kernel(X, linear_tables, dnn_tables, W_out):
    x_flat = X.astype(jnp.int32).reshape(B * F)
    lin_flat = linear_tables.reshape(F * V)
    dnn_flat = dnn_tables.reshape(F * V, D)
    w = W_out.reshape(F, D)

    mesh = plsc.VectorSubcoreMesh(core_axis_name="c", subcore_axis_name="s",
                                  num_cores=NC, num_subcores=NS)
    run = pl.kernel(
        _sc_body,
        out_type=jax.ShapeDtypeStruct((B,), jnp.float32),
        mesh=mesh,
        compiler_params=pltpu.CompilerParams(
            needs_layout_passes=False, use_tc_tiling_on_sc=False),
        scratch_types=[
            pltpu.VMEM((NB * F,), jnp.int32),
            pltpu.VMEM((F * CB,), jnp.int32),
            pltpu.VMEM((F * CB,), jnp.float32),
            pltpu.VMEM((F * CB, D), jnp.float32),
            pltpu.VMEM((F, D), jnp.float32),
            pltpu.VMEM((NB,), jnp.float32),
            pltpu.SemaphoreType.DMA,
            pltpu.SemaphoreType.DMA,
        ],
    )
    out = run(x_flat, lin_flat, dnn_flat, w)
    return out.reshape(B, 1)


# speedup vs baseline: 16.2876x; 2.0454x over previous
import functools

import jax
import jax.numpy as jnp
from jax import lax
from jax.experimental import pallas as pl
from jax.experimental.pallas import tpu as pltpu
from jax.experimental.pallas import tpu_sc as plsc

B = 16384
F = 26
V = 100000
D = 16

NC, NS, L = 2, 16, 16
NW = NC * NS
NB = B // NW
JG = NB // L

VC = 4096


def _fold_body(dnn_ref, lin_ref, w_ref, out_ref):
    f = pl.program_id(0)
    d = dnn_ref[0]
    w = w_ref[f]
    out_ref[0, 0, :] = lin_ref[0, 0, :] + jnp.sum(d * w[:, None], axis=0)


def _sc_body(xt_hbm, comb_hbm, out_hbm, xl, idxb, vals, outl, sem):
    wid = lax.axis_index("s") * NC + lax.axis_index("c")
    base = wid * NB

    pltpu.sync_copy(xt_hbm.at[:, pl.ds(base, NB)], xl)

    def build_f(f, _):
        off = f * V
        for jc in range(JG):
            v = xl[f, pl.ds(jc * L, L)]
            idxb[pl.ds(f * NB + jc * L, L)] = v + off
        return _
    lax.fori_loop(0, F, build_f, None)

    pltpu.async_copy(comb_hbm.at[idxb], vals, sem).wait()

    def group(jg, _):
        acc = jnp.zeros((L,), jnp.float32)
        for f in range(F):
            acc = acc + vals[pl.ds(f * NB + jg * L, L)]
        outl[pl.ds(jg * L, L)] = 1.0 / (1.0 + jnp.exp(-acc))
        return _
    lax.fori_loop(0, JG, group, None)

    pltpu.sync_copy(outl, out_hbm.at[pl.ds(base, NB)])


@jax.jit
def kernel(X, linear_tables, dnn_tables, W_out):
    xt = X.astype(jnp.int32).T
    dnn_t = jnp.transpose(dnn_tables, (0, 2, 1))
    lin_t = jnp.transpose(linear_tables, (0, 2, 1))
    w = W_out.reshape(F, D)

    comb = pl.pallas_call(
        _fold_body,
        grid=(F, pl.cdiv(V, VC)),
        in_specs=[
            pl.BlockSpec((1, D, VC), lambda f, i: (f, 0, i)),
            pl.BlockSpec((1, 1, VC), lambda f, i: (f, 0, i)),
            pl.BlockSpec((F, D), lambda f, i: (0, 0)),
        ],
        out_specs=pl.BlockSpec((1, 1, VC), lambda f, i: (f, 0, i)),
        out_shape=jax.ShapeDtypeStruct((F, 1, V), jnp.float32),
    )(dnn_t, lin_t, w)
    comb_flat = comb.reshape(F * V)

    mesh = plsc.VectorSubcoreMesh(core_axis_name="c", subcore_axis_name="s",
                                  num_cores=NC, num_subcores=NS)
    run = pl.kernel(
        _sc_body,
        out_type=jax.ShapeDtypeStruct((B,), jnp.float32),
        mesh=mesh,
        compiler_params=pltpu.CompilerParams(
            needs_layout_passes=False, use_tc_tiling_on_sc=False),
        scratch_types=[
            pltpu.VMEM((F, NB), jnp.int32),
            pltpu.VMEM((F * NB,), jnp.int32),
            pltpu.VMEM((F * NB,), jnp.float32),
            pltpu.VMEM((NB,), jnp.float32),
            pltpu.SemaphoreType.DMA,
        ],
    )
    out = run(xt, comb_flat)
    return out.reshape(B, 1)


# speedup vs baseline: 79.7915x; 4.8989x over previous
import functools

import jax
import jax.numpy as jnp
from jax import lax
from jax.experimental import pallas as pl
from jax.experimental.pallas import tpu as pltpu
from jax.experimental.pallas import tpu_sc as plsc

B = 16384
F = 26
V = 100000
D = 16

NC, NS, L = 2, 16, 16
NW = NC * NS
NB = B // NW
JG = NB // L

VP = 102400
VC = 51200


def _fold_body(dnn_ref, lin_ref, w_ref, out_ref):
    f = pl.program_id(0)
    d = dnn_ref[0]
    w = w_ref[f]
    dot = jnp.dot(w[None, :], d, preferred_element_type=jnp.float32)
    out_ref[...] = lin_ref[0, 0, :] + dot[0]


def _sc_body(xt_hbm, comb_hbm, out_hbm, xl, idxb, vals, outl, sem):
    wid = lax.axis_index("s") * NC + lax.axis_index("c")
    base = wid * NB

    pltpu.sync_copy(xt_hbm.at[:, pl.ds(base, NB)], xl)

    def build_f(f, _):
        off = f * VP
        for jc in range(JG):
            v = xl[f, pl.ds(jc * L, L)]
            idxb[pl.ds(f * NB + jc * L, L)] = v + off
        return _
    lax.fori_loop(0, F, build_f, None)

    pltpu.async_copy(comb_hbm.at[idxb], vals, sem).wait()

    def group(jg, _):
        acc = jnp.zeros((L,), jnp.float32)
        for f in range(F):
            acc = acc + vals[pl.ds(f * NB + jg * L, L)]
        outl[pl.ds(jg * L, L)] = 1.0 / (1.0 + jnp.exp(-acc))
        return _
    lax.fori_loop(0, JG, group, None)

    pltpu.sync_copy(outl, out_hbm.at[pl.ds(base, NB)])


@jax.jit
def kernel(X, linear_tables, dnn_tables, W_out):
    xt = X.astype(jnp.int32).T
    dnn_t = jnp.transpose(dnn_tables, (0, 2, 1))
    lin_t = jnp.transpose(linear_tables, (0, 2, 1))
    w = W_out.reshape(F, D)

    comb = pl.pallas_call(
        _fold_body,
        grid=(F, VP // VC),
        in_specs=[
            pl.BlockSpec((1, D, VC), lambda f, i: (f, 0, i)),
            pl.BlockSpec((1, 1, VC), lambda f, i: (f, 0, i)),
            pl.BlockSpec((F, D), lambda f, i: (0, 0)),
        ],
        out_specs=pl.BlockSpec((VC,), lambda f, i: (f * (VP // VC) + i,)),
        out_shape=jax.ShapeDtypeStruct((F * VP,), jnp.float32),
    )(dnn_t, lin_t, w)
    comb_flat = comb

    mesh = plsc.VectorSubcoreMesh(core_axis_name="c", subcore_axis_name="s",
                                  num_cores=NC, num_subcores=NS)
    run = pl.kernel(
        _sc_body,
        out_type=jax.ShapeDtypeStruct((B,), jnp.float32),
        mesh=mesh,
        compiler_params=pltpu.CompilerParams(
            needs_layout_passes=False, use_tc_tiling_on_sc=False),
        scratch_types=[
            pltpu.VMEM((F, NB), jnp.int32),
            pltpu.VMEM((F * NB,), jnp.int32),
            pltpu.VMEM((F * NB,), jnp.float32),
            pltpu.VMEM((NB,), jnp.float32),
            pltpu.SemaphoreType.DMA,
        ],
    )
    out = run(xt, comb_flat)
    return out.reshape(B, 1)


# speedup vs baseline: 88.8279x; 1.1133x over previous
import functools

import jax
import jax.numpy as jnp
from jax import lax
from jax.experimental import pallas as pl
from jax.experimental.pallas import tpu as pltpu
from jax.experimental.pallas import tpu_sc as plsc

B = 16384
F = 26
V = 100000
D = 16

NC, NS, L = 2, 16, 16
NW = NC * NS
NB = B // NW
JG = NB // L

VP = 102400
VC = 102400


def _fold_body(dnn_ref, lin_ref, w_ref, out_ref):
    f = pl.program_id(0)
    d = dnn_ref[0]
    w = w_ref[f]
    dot = jnp.dot(w[None, :], d, preferred_element_type=jnp.float32)
    out_ref[...] = lin_ref[0, 0, :] + dot[0]


def _sc_body(xt_hbm, comb_hbm, out_hbm, xl, idxb, vals, outl, sem):
    wid = lax.axis_index("s") * NC + lax.axis_index("c")
    base = wid * NB

    pltpu.sync_copy(xt_hbm.at[:, pl.ds(base, NB)], xl)

    def build_f(f, _):
        off = f * VP
        for jc in range(JG):
            v = xl[f, pl.ds(jc * L, L)]
            idxb[pl.ds(f * NB + jc * L, L)] = v + off
        return _
    lax.fori_loop(0, F, build_f, None)

    pltpu.async_copy(comb_hbm.at[idxb], vals, sem).wait()

    def group(jg, _):
        acc = jnp.zeros((L,), jnp.float32)
        for f in range(F):
            acc = acc + vals[pl.ds(f * NB + jg * L, L)]
        outl[pl.ds(jg * L, L)] = 1.0 / (1.0 + jnp.exp(-acc))
        return _
    lax.fori_loop(0, JG, group, None)

    pltpu.sync_copy(outl, out_hbm.at[pl.ds(base, NB)])


@jax.jit
def kernel(X, linear_tables, dnn_tables, W_out):
    xt = X.astype(jnp.int32).T
    dnn_t = jnp.transpose(dnn_tables, (0, 2, 1))
    lin_t = jnp.transpose(linear_tables, (0, 2, 1))
    w = W_out.reshape(F, D)

    comb = pl.pallas_call(
        _fold_body,
        grid=(F, VP // VC),
        in_specs=[
            pl.BlockSpec((1, D, VC), lambda f, i: (f, 0, i)),
            pl.BlockSpec((1, 1, VC), lambda f, i: (f, 0, i)),
            pl.BlockSpec((F, D), lambda f, i: (0, 0)),
        ],
        out_specs=pl.BlockSpec((VC,), lambda f, i: (f * (VP // VC) + i,)),
        out_shape=jax.ShapeDtypeStruct((F * VP,), jnp.float32),
    )(dnn_t, lin_t, w)
    comb_flat = comb

    mesh = plsc.VectorSubcoreMesh(core_axis_name="c", subcore_axis_name="s",
                                  num_cores=NC, num_subcores=NS)
    run = pl.kernel(
        _sc_body,
        out_type=jax.ShapeDtypeStruct((B,), jnp.float32),
        mesh=mesh,
        compiler_params=pltpu.CompilerParams(
            needs_layout_passes=False, use_tc_tiling_on_sc=False),
        scratch_types=[
            pltpu.VMEM((F, NB), jnp.int32),
            pltpu.VMEM((F * NB,), jnp.int32),
            pltpu.VMEM((F * NB,), jnp.float32),
            pltpu.VMEM((NB,), jnp.float32),
            pltpu.SemaphoreType.DMA,
        ],
    )
    out = run(xt, comb_flat)
    return out.reshape(B, 1)


# speedup vs baseline: 90.3646x; 1.0173x over previous
import functools

import jax
import jax.numpy as jnp
from jax import lax
from jax.experimental import pallas as pl
from jax.experimental.pallas import tpu as pltpu
from jax.experimental.pallas import tpu_sc as plsc

B = 16384
F = 26
V = 100000
D = 16

NC, NS, L = 2, 16, 16
NW = NC * NS
NB = B // NW
JG = NB // L

VP = 102400
FH = F // 2


def _fold_body(dnn_ref, lin_ref, w_ref, out_ref, *, w_base):
    f = pl.program_id(0)
    d = dnn_ref[0]
    w = w_ref[w_base + f]
    dot = jnp.dot(w[None, :], d, preferred_element_type=jnp.float32)
    out_ref[...] = lin_ref[0, 0, :] + dot[0]


def _sc_gather_body(xt_hbm, comb_hbm, *args, f_base, final):
    if final:
        part_hbm, out_hbm, xl, idxb, vals, outl, sem = args
    else:
        out_hbm, xl, idxb, vals, outl, sem = args
    wid = lax.axis_index("s") * NC + lax.axis_index("c")
    base = wid * NB

    pltpu.sync_copy(xt_hbm.at[pl.ds(f_base, FH), pl.ds(base, NB)], xl)

    def build_f(f, _):
        off = f * VP
        for jc in range(JG):
            v = xl[f, pl.ds(jc * L, L)]
            idxb[pl.ds(f * NB + jc * L, L)] = v + off
        return _
    lax.fori_loop(0, FH, build_f, None)

    pltpu.async_copy(comb_hbm.at[idxb], vals, sem).wait()

    if final:
        pltpu.sync_copy(part_hbm.at[pl.ds(base, NB)], outl)

    def group(jg, _):
        acc = jnp.zeros((L,), jnp.float32)
        for f in range(FH):
            acc = acc + vals[pl.ds(f * NB + jg * L, L)]
        if final:
            acc = acc + outl[pl.ds(jg * L, L)]
            acc = 1.0 / (1.0 + jnp.exp(-acc))
        outl[pl.ds(jg * L, L)] = acc
        return _
    lax.fori_loop(0, JG, group, None)

    pltpu.sync_copy(outl, out_hbm.at[pl.ds(base, NB)])


def _make_fold(w_base):
    return pl.pallas_call(
        functools.partial(_fold_body, w_base=w_base),
        grid=(FH, 1),
        in_specs=[
            pl.BlockSpec((1, D, VP), lambda f, i: (w_base + f, 0, i)),
            pl.BlockSpec((1, 1, VP), lambda f, i: (w_base + f, 0, i)),
            pl.BlockSpec((F, D), lambda f, i: (0, 0)),
        ],
        out_specs=pl.BlockSpec((VP,), lambda f, i: (f,)),
        out_shape=jax.ShapeDtypeStruct((FH * VP,), jnp.float32),
    )


def _make_gather(f_base, final):
    mesh = plsc.VectorSubcoreMesh(core_axis_name="c", subcore_axis_name="s",
                                  num_cores=NC, num_subcores=NS)
    return pl.kernel(
        functools.partial(_sc_gather_body, f_base=f_base, final=final),
        out_type=jax.ShapeDtypeStruct((B,), jnp.float32),
        mesh=mesh,
        compiler_params=pltpu.CompilerParams(
            needs_layout_passes=False, use_tc_tiling_on_sc=False),
        scratch_types=[
            pltpu.VMEM((FH, NB), jnp.int32),
            pltpu.VMEM((FH * NB,), jnp.int32),
            pltpu.VMEM((FH * NB,), jnp.float32),
            pltpu.VMEM((NB,), jnp.float32),
            pltpu.SemaphoreType.DMA,
        ],
    )


@jax.jit
def kernel(X, linear_tables, dnn_tables, W_out):
    xt = X.astype(jnp.int32).T
    dnn_t = jnp.transpose(dnn_tables, (0, 2, 1))
    lin_t = jnp.transpose(linear_tables, (0, 2, 1))
    w = W_out.reshape(F, D)

    comb0 = _make_fold(0)(dnn_t, lin_t, w)
    part = _make_gather(0, final=False)(xt, comb0)
    comb1 = _make_fold(FH)(dnn_t, lin_t, w)
    out = _make_gather(FH, final=True)(xt, comb1, part)
    return out.reshape(B, 1)
